# Initial kernel scaffold; baseline (speedup 1.0000x reference)
import jax
import jax.numpy as jnp
from jax.experimental import pallas as pl


def kernel(x, adj1_indices, adj1_values, adj2_indices, adj2_values, W1):
    raise NotImplementedError("write your pallas kernel here")


# speedup vs baseline: 4.5561x; 4.5561x over previous
import jax
import jax.numpy as jnp
from jax import lax
from jax.experimental import pallas as pl
from jax.experimental.pallas import tpu as pltpu
from jax.experimental.pallas import tpu_sc as plsc

N_NODES = 10000
DIM = 128
N_EDGES = 320000
NUM_CORES = 2
NUM_SUBCORES = 16
LANES = 16

CHUNK = 128
NCH = N_EDGES // CHUNK
ROWS_A = 624
ROWS_LAST = N_NODES - ROWS_A * (NUM_SUBCORES - 1)


def _matmul_body(x_ref, w_ref, o_ref):
    o_ref[...] = lax.dot_general(
        x_ref[...], w_ref[...], (((1,), (1,)), ((), ())),
        preferred_element_type=jnp.float32)


def _h0_matmul(x, W1):
    return pl.pallas_call(
        _matmul_body,
        grid=(10,),
        in_specs=[pl.BlockSpec((1000, DIM), lambda i: (i, 0)),
                  pl.BlockSpec((DIM, DIM), lambda i: (0, 0))],
        out_specs=pl.BlockSpec((1000, DIM), lambda i: (i, 0)),
        out_shape=jax.ShapeDtypeStruct((N_NODES, DIM), jnp.float32),
    )(x, W1)


def _spmm_body(h0_hbm, dst_hbm, src_hbm, val_hbm, zeros_hbm, out_hbm,
               dst_v, src_v, val_v, rows_v, acc_sh, sem):
    c = lax.axis_index("c")
    s = lax.axis_index("s")
    row0 = s * ROWS_A
    last = NUM_SUBCORES - 1

    @pl.when(s < last)
    def _():
        pltpu.sync_copy(zeros_hbm.at[pl.ds(0, ROWS_A)],
                        acc_sh.at[pl.ds(row0, ROWS_A)])

    @pl.when(s == last)
    def _():
        pltpu.sync_copy(zeros_hbm, acc_sh.at[pl.ds(last * ROWS_A, ROWS_LAST)])

    plsc.subcore_barrier()

    n_mine = jnp.where(s < NCH % NUM_SUBCORES,
                       NCH // NUM_SUBCORES + 1, NCH // NUM_SUBCORES)

    def chunk_body(i, carry):
        off = c * N_EDGES + (s + i * NUM_SUBCORES) * CHUNK
        pltpu.sync_copy(dst_hbm.at[pl.ds(off, CHUNK)], dst_v)
        pltpu.sync_copy(src_hbm.at[pl.ds(off, CHUNK)], src_v)
        pltpu.sync_copy(val_hbm.at[pl.ds(off, CHUNK)], val_v)
        pltpu.async_copy(h0_hbm.at[src_v], rows_v, sem).wait()
        for g in range(CHUNK // LANES):
            v16 = val_v[pl.ds(g * LANES, LANES)]
            for l in range(LANES):
                e = g * LANES + l
                vv = jnp.broadcast_to(v16[l], (LANES,))
                for j in range(DIM // LANES):
                    sl = pl.ds(j * LANES, LANES)
                    rows_v[e, sl] = rows_v[e, sl] * vv
        pltpu.sync_copy(rows_v, acc_sh.at[dst_v], add=True)
        return carry

    lax.fori_loop(0, n_mine, chunk_body, 0)
    plsc.subcore_barrier()

    @pl.when(s < last)
    def _():
        pltpu.sync_copy(acc_sh.at[pl.ds(row0, ROWS_A)],
                        out_hbm.at[c, pl.ds(row0, ROWS_A)])

    @pl.when(s == last)
    def _():
        pltpu.sync_copy(acc_sh.at[pl.ds(last * ROWS_A, ROWS_LAST)],
                        out_hbm.at[c, pl.ds(last * ROWS_A, ROWS_LAST)])


def _spmm_both(h0, dst_all, src_all, val_all, zeros):
    mesh = plsc.VectorSubcoreMesh(core_axis_name="c", subcore_axis_name="s")
    return pl.kernel(
        _spmm_body,
        out_type=jax.ShapeDtypeStruct((NUM_CORES, N_NODES, DIM), jnp.float32),
        mesh=mesh,
        scratch_types=[
            pltpu.VMEM((CHUNK,), jnp.int32),
            pltpu.VMEM((CHUNK,), jnp.int32),
            pltpu.VMEM((CHUNK,), jnp.float32),
            pltpu.VMEM((CHUNK, DIM), jnp.float32),
            pltpu.VMEM_SHARED((N_NODES, DIM), jnp.float32),
            pltpu.SemaphoreType.DMA,
        ],
    )(h0, dst_all, src_all, val_all, zeros)


def kernel(x, adj1_indices, adj1_values, adj2_indices, adj2_values, W1):
    h0 = _h0_matmul(x, W1)
    i1 = adj1_indices.astype(jnp.int32)
    i2 = adj2_indices.astype(jnp.int32)
    dst_all = jnp.concatenate([i1[0], i2[0]])
    src_all = jnp.concatenate([i1[1], i2[1]])
    val_all = jnp.concatenate([adj1_values, adj2_values])
    zeros = jnp.zeros((ROWS_LAST, DIM), jnp.float32)
    hops = _spmm_both(h0, dst_all, src_all, val_all, zeros)
    return jnp.concatenate([h0, hops[0], hops[1]], axis=1)
